# baseline (speedup 1.0000x reference)
import functools

import jax
import jax.numpy as jnp
from jax import lax
from jax.experimental import pallas as pl
from jax.experimental.pallas import tpu as pltpu
from jax.experimental.pallas import tpu_sc as plsc

VOCAB = 1000000
DIM = 64
NC, NS = 2, 16
NW = NC * NS
B = 4096 * 200
NPAIR = B // 2
PPW = NPAIR // NW
SUB = 128
CHUNK = 256
KROW = CHUNK // SUB
NCHUNK = PPW // CHUNK
IDXROWS = PPW // SUB


def _embed_lookup(xe2d, xo2d, table):
    mesh = plsc.VectorSubcoreMesh(core_axis_name="c", subcore_axis_name="s")

    @functools.partial(
        pl.kernel,
        out_type=jax.ShapeDtypeStruct((NPAIR, 128), jnp.float32),
        mesh=mesh,
        scratch_types=[
            pltpu.VMEM((IDXROWS, SUB), jnp.int32),
            pltpu.VMEM((IDXROWS, SUB), jnp.int32),
            pltpu.VMEM((2, CHUNK, DIM), jnp.float32),
            pltpu.VMEM((2, CHUNK, DIM), jnp.float32),
            pltpu.SemaphoreType.DMA,
            pltpu.SemaphoreType.DMA,
            pltpu.SemaphoreType.DMA,
        ],
        compiler_params=pltpu.CompilerParams(use_tc_tiling_on_sc=False),
    )
    def body(xe_hbm, xo_hbm, table_hbm, out_hbm, idxe_v, idxo_v, ebuf, obuf,
             gsem0, gsem1, osem):
        wid = lax.axis_index("s") * NC + lax.axis_index("c")
        row0 = wid * IDXROWS
        gsems = (gsem0, gsem1)

        pltpu.sync_copy(xe_hbm.at[pl.ds(row0, IDXROWS)], idxe_v)
        pltpu.sync_copy(xo_hbm.at[pl.ds(row0, IDXROWS)], idxo_v)

        def gather_copies(i, b):
            copies = []
            for j in range(KROW):
                sl = pl.ds(j * SUB, SUB)
                copies.append(pltpu.make_async_copy(
                    table_hbm.at[idxe_v.at[i * KROW + j]],
                    ebuf.at[b].at[sl],
                    gsems[b],
                ))
                copies.append(pltpu.make_async_copy(
                    table_hbm.at[idxo_v.at[i * KROW + j]],
                    obuf.at[b].at[sl],
                    gsems[b],
                ))
            return copies

        def fire_gathers(i, b):
            for c in gather_copies(i, b):
                c.start()

        def drain_gathers(i, b):
            for c in gather_copies(i, b):
                c.wait()

        def out_copies(i, b):
            dst = out_hbm.at[pl.ds(wid * PPW + i * CHUNK, CHUNK)]
            return [
                pltpu.make_async_copy(ebuf.at[b], dst.at[:, pl.ds(0, DIM)],
                                      osem),
                pltpu.make_async_copy(obuf.at[b], dst.at[:, pl.ds(DIM, DIM)],
                                      osem),
            ]

        fire_gathers(0, 0)

        def outer(t, carry):
            for b in range(2):
                i = t * 2 + b

                @pl.when(i > 0)
                def _():
                    for c in out_copies(i - 1, 1 - b):
                        c.wait()

                @pl.when(i + 1 < NCHUNK)
                def _():
                    fire_gathers(i + 1, 1 - b)

                drain_gathers(i, b)
                for c in out_copies(i, b):
                    c.start()
            return carry

        lax.fori_loop(0, NCHUNK // 2, outer, 0)
        for c in out_copies(NCHUNK - 1, 1):
            c.wait()

    return body(xe2d, xo2d, table)


def kernel(x, table):
    xp = x.reshape(NPAIR, 2).astype(jnp.int32)
    xe2d = xp[:, 0].reshape(NPAIR // SUB, SUB)
    xo2d = xp[:, 1].reshape(NPAIR // SUB, SUB)
    out = _embed_lookup(xe2d, xo2d, table)
    return out.reshape(4096, 200, DIM)


# speedup vs baseline: 1.1739x; 1.1739x over previous
import functools

import jax
import jax.numpy as jnp
from jax import lax
from jax.experimental import pallas as pl
from jax.experimental.pallas import tpu as pltpu
from jax.experimental.pallas import tpu_sc as plsc

VOCAB = 1000000
DIM = 64
NC, NS = 2, 16
NW = NC * NS
B = 4096 * 200
BPW = B // NW
SUB = 128
K = 4
CHUNK = SUB * K
NCHUNK = BPW // CHUNK
IDXROWS = BPW // SUB


def _embed_lookup(x2d, table):
    mesh = plsc.VectorSubcoreMesh(core_axis_name="c", subcore_axis_name="s")

    @functools.partial(
        pl.kernel,
        out_type=jax.ShapeDtypeStruct((B, DIM), jnp.float32),
        mesh=mesh,
        scratch_types=[
            pltpu.VMEM((IDXROWS, SUB), jnp.int32),
            pltpu.VMEM((2, CHUNK, DIM), jnp.float32),
            pltpu.SemaphoreType.DMA,
            pltpu.SemaphoreType.DMA,
            pltpu.SemaphoreType.DMA,
        ],
        compiler_params=pltpu.CompilerParams(
            use_tc_tiling_on_sc=False, skip_device_barrier=True),
    )
    def body(x_hbm, table_hbm, out_hbm, idx_v, rows_v, gsem0, gsem1, osem):
        wid = lax.axis_index("s") * NC + lax.axis_index("c")
        row0 = wid * IDXROWS
        gsems = (gsem0, gsem1)
        table_rows = table_hbm

        pltpu.sync_copy(x_hbm.at[pl.ds(row0, IDXROWS)], idx_v)

        def fire_gathers(i, b):
            for j in range(K):
                pltpu.async_copy(
                    table_rows.at[idx_v.at[i * K + j]],
                    rows_v.at[b].at[pl.ds(j * SUB, SUB)],
                    gsems[b],
                )

        def drain_gathers(i, b):
            for j in range(K):
                pltpu.make_async_copy(
                    table_rows.at[idx_v.at[i * K + j]],
                    rows_v.at[b].at[pl.ds(j * SUB, SUB)],
                    gsems[b],
                ).wait()

        def fire_out(i, b):
            pltpu.async_copy(
                rows_v.at[b],
                out_hbm.at[pl.ds(row0 * SUB + i * CHUNK, CHUNK)],
                osem,
            )

        def drain_out(i, b):
            pltpu.make_async_copy(
                rows_v.at[b],
                out_hbm.at[pl.ds(row0 * SUB + i * CHUNK, CHUNK)],
                osem,
            ).wait()

        fire_gathers(0, 0)

        def outer(t, carry):
            for b in range(2):
                i = t * 2 + b

                @pl.when(i > 0)
                def _():
                    drain_out(i - 1, 1 - b)

                @pl.when(i + 1 < NCHUNK)
                def _():
                    fire_gathers(i + 1, 1 - b)

                drain_gathers(i, b)
                fire_out(i, b)
            return carry

        lax.fori_loop(0, NCHUNK // 2, outer, 0)
        drain_out(NCHUNK - 1, 1)

    return body(x2d, table)


def kernel(x, table):
    x2d = x.reshape(B // SUB, SUB).astype(jnp.int32)
    out = _embed_lookup(x2d, table)
    return out.reshape(4096, 200, DIM)
